# baseline (speedup 1.0000x reference)
import functools

import jax
import jax.numpy as jnp
from jax import lax
from jax.experimental import pallas as pl
from jax.experimental.pallas import tpu as pltpu
from jax.experimental.pallas import tpu_sc as plsc

N = 10000
E = 320000
D_IN = 128
H1 = 64
H2 = 32

NC = 2
NS = 16
NW = NC * NS
E_PER_W = E // NW
N_PAD = 10240
ROWS_PER_TILE = N_PAD // NS
CHUNK = 1000
NCHUNK = E_PER_W // CHUNK


def _edge_serial(table, acc, idx_v, r0, gs0, nchunk):
    def body(i, carry):
        pltpu.async_copy(table.at[idx_v.at[0, i]], r0, gs0).wait()
        pltpu.sync_copy(r0, acc.at[idx_v.at[1, i]], add=True)
        return carry

    lax.fori_loop(0, nchunk, body, 0)


def _edge_pipelined(table, acc, idx_v, r0, r1, gs0, gs1, ss0, ss1, nchunk):
    def body(s, carry):
        i0 = 2 * s
        i1 = i0 + 1

        @pl.when(s > 0)
        def _():
            pltpu.make_async_copy(r0, acc.at[idx_v.at[1, i0]], ss0).wait()

        g0 = pltpu.async_copy(table.at[idx_v.at[0, i0]], r0, gs0)

        @pl.when(s > 0)
        def _():
            pltpu.make_async_copy(r1, acc.at[idx_v.at[1, i1]], ss1).wait()

        g1 = pltpu.async_copy(table.at[idx_v.at[0, i1]], r1, gs1)
        g0.wait()
        pltpu.async_copy(r0, acc.at[idx_v.at[1, i0]], ss0, add=True)
        g1.wait()
        pltpu.async_copy(r1, acc.at[idx_v.at[1, i1]], ss1, add=True)
        return carry

    lax.fori_loop(0, nchunk // 2, body, 0)
    pltpu.make_async_copy(r0, acc.at[idx_v.at[1, 0]], ss0).wait()
    pltpu.make_async_copy(r1, acc.at[idx_v.at[1, 1]], ss1).wait()


def _sc_layer_call(tables, edges, zeros, d, pipelined):
    mesh = plsc.VectorSubcoreMesh(core_axis_name="c", subcore_axis_name="s")
    osh = jax.ShapeDtypeStruct((NC, N_PAD, d), jnp.float32)
    nbuf = 2 if pipelined else 1

    @functools.partial(
        pl.kernel,
        mesh=mesh,
        compiler_params=pltpu.CompilerParams(use_tc_tiling_on_sc=False),
        out_type=[osh, osh, osh, osh],
        scratch_types=[
            pltpu.VMEM((2, NCHUNK, CHUNK), jnp.int32),
            [pltpu.VMEM((CHUNK, d), jnp.float32)] * nbuf,
            pltpu.VMEM_SHARED((N_PAD, d), jnp.float32),
            [pltpu.SemaphoreType.DMA] * (4 if pipelined else 1),
        ],
    )
    def k(t0, t1, t2, t3, i0, i1, i2, i3, zeros_hbm,
          o0, o1, o2, o3,
          idx_v, rbufs, acc, sems):
        c = lax.axis_index("c")
        s = lax.axis_index("s")
        w = c * NS + s
        row0 = s * ROWS_PER_TILE
        rows = pl.ds(row0, ROWS_PER_TILE)
        pltpu.sync_copy(zeros_hbm.at[rows], acc.at[rows])
        plsc.subcore_barrier()
        for t, (tab, ih, out) in enumerate(
                zip((t0, t1, t2, t3), (i0, i1, i2, i3), (o0, o1, o2, o3))):
            pltpu.sync_copy(ih.at[1, w], idx_v.at[0])
            pltpu.sync_copy(ih.at[0, w], idx_v.at[1])
            if pipelined:
                _edge_pipelined(tab, acc, idx_v, rbufs[0], rbufs[1],
                                sems[0], sems[1], sems[2], sems[3], NCHUNK)
            else:
                _edge_serial(tab, acc, idx_v, rbufs[0], sems[0], NCHUNK)
            plsc.subcore_barrier()
            pltpu.sync_copy(acc.at[rows], out.at[c, rows])
            if t < 3:
                pltpu.sync_copy(zeros_hbm.at[rows], acc.at[rows])
            plsc.subcore_barrier()

    return k(*tables, *edges, zeros)


def _bd(w, k):
    return jax.scipy.linalg.block_diag(*([w] * k))


def _gnorm(x, g):
    width = x.shape[1] // g
    parts = []
    for i in range(g):
        xs = x[:, i * width:(i + 1) * width]
        n = jnp.sqrt(jnp.maximum(
            jnp.sum(xs * xs, axis=1, keepdims=True), 1e-12))
        parts.append(xs / n)
    return jnp.concatenate(parts, axis=1)


def _t1_body(f0, f1, w00, w01, w10, w11, h00, h01, h10, h11):
    rb = f0.shape[0]
    a = f0[...].reshape(rb // 2, 2 * D_IN)
    b = f1[...].reshape(rb // 2, 2 * D_IN)
    h00[...] = jnp.dot(a, w00[...], preferred_element_type=jnp.float32)
    h01[...] = jnp.dot(b, w01[...], preferred_element_type=jnp.float32)
    h10[...] = jnp.dot(a, w10[...], preferred_element_type=jnp.float32)
    h11[...] = jnp.dot(b, w11[...], preferred_element_type=jnp.float32)


def _t1(f0, f1, w00, w01, w10, w11):
    rb = 2000
    fs = pl.BlockSpec((rb, D_IN), lambda i: (i, 0))
    ws = pl.BlockSpec((2 * D_IN, 2 * H1), lambda i: (0, 0))
    os = pl.BlockSpec((rb // 2, 2 * H1), lambda i: (i, 0))
    sh = jax.ShapeDtypeStruct((N // 2, 2 * H1), jnp.float32)
    return pl.pallas_call(
        _t1_body,
        grid=(N // rb,),
        in_specs=[fs, fs, ws, ws, ws, ws],
        out_specs=[os, os, os, os],
        out_shape=[sh, sh, sh, sh],
    )(f0, f1, w00, w01, w10, w11)


def _t2_body(a00, a01, a10, a11, w00, w01, w10, w11, g00, g01, g10, g11):
    rb = a00.shape[1]
    h0 = jax.nn.relu(_gnorm(a00[0] + a00[1], 2) + _gnorm(a01[0] + a01[1], 2))
    h1 = jax.nn.relu(_gnorm(a10[0] + a10[1], 2) + _gnorm(a11[0] + a11[1], 2))
    h0 = h0.reshape(rb // 2, 4 * H1)
    h1 = h1.reshape(rb // 2, 4 * H1)
    g00[...] = jnp.dot(h0, w00[...], preferred_element_type=jnp.float32)
    g01[...] = jnp.dot(h1, w01[...], preferred_element_type=jnp.float32)
    g10[...] = jnp.dot(h0, w10[...], preferred_element_type=jnp.float32)
    g11[...] = jnp.dot(h1, w11[...], preferred_element_type=jnp.float32)


def _t2(a00p, a01p, a10p, a11p, w00, w01, w10, w11):
    r2 = N_PAD // 2
    rb = 2560
    asp = pl.BlockSpec((NC, rb, 2 * H1), lambda i: (0, i, 0))
    ws = pl.BlockSpec((4 * H1, 4 * H2), lambda i: (0, 0))
    os = pl.BlockSpec((rb // 2, 4 * H2), lambda i: (i, 0))
    sh = jax.ShapeDtypeStruct((N_PAD // 4, 4 * H2), jnp.float32)
    return pl.pallas_call(
        _t2_body,
        grid=(r2 // rb,),
        in_specs=[asp, asp, asp, asp, ws, ws, ws, ws],
        out_specs=[os, os, os, os],
        out_shape=[sh, sh, sh, sh],
    )(a00p, a01p, a10p, a11p, w00, w01, w10, w11)


def _t3_body(b00, b01, b10, b11, e0, e1):
    e0[...] = _gnorm(b00[0] + b00[1], 4) + _gnorm(b01[0] + b01[1], 4)
    e1[...] = _gnorm(b10[0] + b10[1], 4) + _gnorm(b11[0] + b11[1], 4)


def _t3(b00p, b01p, b10p, b11p):
    r4 = N_PAD // 4
    rb = 512
    bsp = pl.BlockSpec((NC, rb, 4 * H2), lambda i: (0, i, 0))
    os = pl.BlockSpec((rb, 4 * H2), lambda i: (i, 0))
    sh = jax.ShapeDtypeStruct((r4, 4 * H2), jnp.float32)
    return pl.pallas_call(
        _t3_body,
        grid=(r4 // rb,),
        in_specs=[bsp, bsp, bsp, bsp],
        out_specs=[os, os],
        out_shape=[sh, sh],
    )(b00p, b01p, b10p, b11p)


def kernel(feat_0, feat_1, ei_00, ei_01, ei_10, ei_11,
           W1_00, W1_01, W1_10, W1_11,
           W2_00, W2_01, W2_10, W2_11):
    edges = [e.astype(jnp.int32).reshape(2, NW, NCHUNK, CHUNK)
             for e in (ei_00, ei_01, ei_10, ei_11)]
    z1 = jnp.zeros((N_PAD, H1), jnp.float32)
    z2 = jnp.zeros((N_PAD, H2), jnp.float32)

    h00r, h01r, h10r, h11r = _t1(
        feat_0, feat_1,
        _bd(W1_00, 2), _bd(W1_01, 2), _bd(W1_10, 2), _bd(W1_11, 2))

    a00, a01, a10, a11 = _sc_layer_call(
        tuple(h.reshape(N, H1) for h in (h00r, h01r, h10r, h11r)),
        edges, z1, H1, False)

    g00q, g01q, g10q, g11q = _t2(
        a00.reshape(NC, N_PAD // 2, 2 * H1),
        a01.reshape(NC, N_PAD // 2, 2 * H1),
        a10.reshape(NC, N_PAD // 2, 2 * H1),
        a11.reshape(NC, N_PAD // 2, 2 * H1),
        _bd(W2_00, 4), _bd(W2_01, 4), _bd(W2_10, 4), _bd(W2_11, 4))

    b00, b01, b10, b11 = _sc_layer_call(
        tuple(g.reshape(N_PAD, H2) for g in (g00q, g01q, g10q, g11q)),
        edges, z2, H2, True)

    e0q, e1q = _t3(
        b00.reshape(NC, N_PAD // 4, 4 * H2),
        b01.reshape(NC, N_PAD // 4, 4 * H2),
        b10.reshape(NC, N_PAD // 4, 4 * H2),
        b11.reshape(NC, N_PAD // 4, 4 * H2))

    e0 = e0q.reshape(N_PAD, H2)[:N]
    e1 = e1q.reshape(N_PAD, H2)[:N]
    return jnp.concatenate([e0, e1], axis=0)


# speedup vs baseline: 1.0254x; 1.0254x over previous
import functools

import jax
import jax.numpy as jnp
from jax import lax
from jax.experimental import pallas as pl
from jax.experimental.pallas import tpu as pltpu
from jax.experimental.pallas import tpu_sc as plsc

N = 10000
E = 320000
D_IN = 128
H1 = 64
H2 = 32

NC = 2
NS = 16
NW = NC * NS
E_PER_W = E // NW
N_PAD = 10240
ROWS_PER_TILE = N_PAD // NS
CHUNK = 1000
NCHUNK = E_PER_W // CHUNK


def _edge_serial(table, acc, idx_v, r0, gs0, nchunk):
    def body(i, carry):
        pltpu.async_copy(table.at[idx_v.at[0, i]], r0, gs0).wait()
        pltpu.sync_copy(r0, acc.at[idx_v.at[1, i]], add=True)
        return carry

    lax.fori_loop(0, nchunk, body, 0)


def _edge_pipelined(table, acc, idx_v, r0, r1, gs0, gs1, ss0, ss1, nchunk):
    def body(s, carry):
        i0 = 2 * s
        i1 = i0 + 1

        @pl.when(s > 0)
        def _():
            pltpu.make_async_copy(r0, acc.at[idx_v.at[1, i0]], ss0).wait()

        g0 = pltpu.async_copy(table.at[idx_v.at[0, i0]], r0, gs0)

        @pl.when(s > 0)
        def _():
            pltpu.make_async_copy(r1, acc.at[idx_v.at[1, i1]], ss1).wait()

        g1 = pltpu.async_copy(table.at[idx_v.at[0, i1]], r1, gs1)
        g0.wait()
        pltpu.async_copy(r0, acc.at[idx_v.at[1, i0]], ss0, add=True)
        g1.wait()
        pltpu.async_copy(r1, acc.at[idx_v.at[1, i1]], ss1, add=True)
        return carry

    lax.fori_loop(0, nchunk // 2, body, 0)
    pltpu.make_async_copy(r0, acc.at[idx_v.at[1, 0]], ss0).wait()
    pltpu.make_async_copy(r1, acc.at[idx_v.at[1, 1]], ss1).wait()


def _sc_pair_call(tables, edges, zeros, d, pipelined):
    mesh = plsc.VectorSubcoreMesh(core_axis_name="c", subcore_axis_name="s")
    osh = jax.ShapeDtypeStruct((NC, N_PAD, d), jnp.float32)
    nbuf = 2 if pipelined else 1

    @functools.partial(
        pl.kernel,
        mesh=mesh,
        compiler_params=pltpu.CompilerParams(use_tc_tiling_on_sc=False),
        out_type=[osh, osh],
        scratch_types=[
            pltpu.VMEM((2, NCHUNK, CHUNK), jnp.int32),
            [pltpu.VMEM((CHUNK, d), jnp.float32)] * nbuf,
            pltpu.VMEM_SHARED((N_PAD, d), jnp.float32),
            [pltpu.SemaphoreType.DMA] * (4 if pipelined else 1),
        ],
    )
    def k(t0, t1, i0, i1, zeros_hbm, o0, o1,
          idx_v, rbufs, acc, sems):
        c = lax.axis_index("c")
        s = lax.axis_index("s")
        w = c * NS + s
        row0 = s * ROWS_PER_TILE
        rows = pl.ds(row0, ROWS_PER_TILE)
        pltpu.sync_copy(zeros_hbm.at[rows], acc.at[rows])
        plsc.subcore_barrier()
        for t, (tab, ih, out) in enumerate(
                zip((t0, t1), (i0, i1), (o0, o1))):
            pltpu.sync_copy(ih.at[1, w], idx_v.at[0])
            pltpu.sync_copy(ih.at[0, w], idx_v.at[1])
            if pipelined:
                _edge_pipelined(tab, acc, idx_v, rbufs[0], rbufs[1],
                                sems[0], sems[1], sems[2], sems[3], NCHUNK)
            else:
                _edge_serial(tab, acc, idx_v, rbufs[0], sems[0], NCHUNK)
            plsc.subcore_barrier()
            pltpu.sync_copy(acc.at[rows], out.at[c, rows])
            if t < 1:
                pltpu.sync_copy(zeros_hbm.at[rows], acc.at[rows])
            plsc.subcore_barrier()

    return k(*tables, *edges, zeros)


def _bd(w, k):
    return jax.scipy.linalg.block_diag(*([w] * k))


def _gnorm(x, g):
    width = x.shape[1] // g
    parts = []
    for i in range(g):
        xs = x[:, i * width:(i + 1) * width]
        n = jnp.sqrt(jnp.maximum(
            jnp.sum(xs * xs, axis=1, keepdims=True), 1e-12))
        parts.append(xs / n)
    return jnp.concatenate(parts, axis=1)


def _t1_body(f0, f1, w00, w01, w10, w11, h00, h01, h10, h11):
    rb = f0.shape[0]
    a = f0[...].reshape(rb // 2, 2 * D_IN)
    b = f1[...].reshape(rb // 2, 2 * D_IN)
    h00[...] = jnp.dot(a, w00[...], preferred_element_type=jnp.float32)
    h01[...] = jnp.dot(b, w01[...], preferred_element_type=jnp.float32)
    h10[...] = jnp.dot(a, w10[...], preferred_element_type=jnp.float32)
    h11[...] = jnp.dot(b, w11[...], preferred_element_type=jnp.float32)


def _t1(f0, f1, w00, w01, w10, w11):
    rb = 2000
    fs = pl.BlockSpec((rb, D_IN), lambda i: (i, 0))
    ws = pl.BlockSpec((2 * D_IN, 2 * H1), lambda i: (0, 0))
    os = pl.BlockSpec((rb // 2, 2 * H1), lambda i: (i, 0))
    sh = jax.ShapeDtypeStruct((N // 2, 2 * H1), jnp.float32)
    return pl.pallas_call(
        _t1_body,
        grid=(N // rb,),
        in_specs=[fs, fs, ws, ws, ws, ws],
        out_specs=[os, os, os, os],
        out_shape=[sh, sh, sh, sh],
    )(f0, f1, w00, w01, w10, w11)


def _t2h_body(ax, ay, wa, wb, ga, gb):
    rb = ax.shape[1]
    h = jax.nn.relu(_gnorm(ax[0] + ax[1], 2) + _gnorm(ay[0] + ay[1], 2))
    h = h.reshape(rb // 2, 4 * H1)
    ga[...] = jnp.dot(h, wa[...], preferred_element_type=jnp.float32)
    gb[...] = jnp.dot(h, wb[...], preferred_element_type=jnp.float32)


def _t2h(axp, ayp, wa, wb):
    r2 = N_PAD // 2
    rb = 2560
    asp = pl.BlockSpec((NC, rb, 2 * H1), lambda i: (0, i, 0))
    ws = pl.BlockSpec((4 * H1, 4 * H2), lambda i: (0, 0))
    os = pl.BlockSpec((rb // 2, 4 * H2), lambda i: (i, 0))
    sh = jax.ShapeDtypeStruct((N_PAD // 4, 4 * H2), jnp.float32)
    return pl.pallas_call(
        _t2h_body,
        grid=(r2 // rb,),
        in_specs=[asp, asp, ws, ws],
        out_specs=[os, os],
        out_shape=[sh, sh],
    )(axp, ayp, wa, wb)


def _t3_body(b00, b01, b10, b11, e0, e1):
    e0[...] = _gnorm(b00[0] + b00[1], 4) + _gnorm(b01[0] + b01[1], 4)
    e1[...] = _gnorm(b10[0] + b10[1], 4) + _gnorm(b11[0] + b11[1], 4)


def _t3(b00p, b01p, b10p, b11p):
    r4 = N_PAD // 4
    rb = 512
    bsp = pl.BlockSpec((NC, rb, 4 * H2), lambda i: (0, i, 0))
    os = pl.BlockSpec((rb, 4 * H2), lambda i: (i, 0))
    sh = jax.ShapeDtypeStruct((r4, 4 * H2), jnp.float32)
    return pl.pallas_call(
        _t3_body,
        grid=(r4 // rb,),
        in_specs=[bsp, bsp, bsp, bsp],
        out_specs=[os, os],
        out_shape=[sh, sh],
    )(b00p, b01p, b10p, b11p)


def kernel(feat_0, feat_1, ei_00, ei_01, ei_10, ei_11,
           W1_00, W1_01, W1_10, W1_11,
           W2_00, W2_01, W2_10, W2_11):
    edges = [e.astype(jnp.int32).reshape(2, NW, NCHUNK, CHUNK)
             for e in (ei_00, ei_01, ei_10, ei_11)]
    z1 = jnp.zeros((N_PAD, H1), jnp.float32)
    z2 = jnp.zeros((N_PAD, H2), jnp.float32)

    h00r, h01r, h10r, h11r = _t1(
        feat_0, feat_1,
        _bd(W1_00, 2), _bd(W1_01, 2), _bd(W1_10, 2), _bd(W1_11, 2))

    a00, a01 = _sc_pair_call(
        (h00r.reshape(N, H1), h01r.reshape(N, H1)),
        (edges[0], edges[1]), z1, H1, False)
    a10, a11 = _sc_pair_call(
        (h10r.reshape(N, H1), h11r.reshape(N, H1)),
        (edges[2], edges[3]), z1, H1, False)

    g00q, g10q = _t2h(
        a00.reshape(NC, N_PAD // 2, 2 * H1),
        a01.reshape(NC, N_PAD // 2, 2 * H1),
        _bd(W2_00, 4), _bd(W2_10, 4))
    g01q, g11q = _t2h(
        a10.reshape(NC, N_PAD // 2, 2 * H1),
        a11.reshape(NC, N_PAD // 2, 2 * H1),
        _bd(W2_01, 4), _bd(W2_11, 4))

    b00, b10 = _sc_pair_call(
        (g00q.reshape(N_PAD, H2), g10q.reshape(N_PAD, H2)),
        (edges[0], edges[2]), z2, H2, True)
    b01, b11 = _sc_pair_call(
        (g01q.reshape(N_PAD, H2), g11q.reshape(N_PAD, H2)),
        (edges[1], edges[3]), z2, H2, True)

    e0q, e1q = _t3(
        b00.reshape(NC, N_PAD // 4, 4 * H2),
        b01.reshape(NC, N_PAD // 4, 4 * H2),
        b10.reshape(NC, N_PAD // 4, 4 * H2),
        b11.reshape(NC, N_PAD // 4, 4 * H2))

    e0 = e0q.reshape(N_PAD, H2)[:N]
    e1 = e1q.reshape(N_PAD, H2)[:N]
    return jnp.concatenate([e0, e1], axis=0)
